# baseline (speedup 1.0000x reference)
import jax
import jax.numpy as jnp
from jax import lax
from jax.experimental import pallas as pl
from jax.experimental.pallas import tpu as pltpu
from jax.experimental.pallas import tpu_sc as plsc

_S = 4096
_R = 26
_D = 64
_NC, _NS = 2, 16
_NW = _NC * _NS
_SB = _S // _NW
_RC = 13
_CHUNK = _RC * _SB


def _embed_body(xt_hbm, table_hbm, out_hbm, idx_v, rows_v, outa_v, outb_v,
                gsem, wsem_a, wsem_b, isem):
    w = lax.axis_index("s") * _NC + lax.axis_index("c")
    s0 = w * _SB
    lane = jax.lax.iota(jnp.int32, 16)
    for c in range(2):
        ihs = [
            pltpu.async_copy(
                xt_hbm.at[_RC * c + j, pl.ds(s0, _SB)],
                idx_v.at[pl.ds(_SB * j, _SB)],
                isem,
            )
            for j in range(_RC)
        ]
        for h in ihs:
            h.wait()
        pltpu.async_copy(table_hbm.at[idx_v], rows_v, gsem).wait()

        cols = [jnp.full((16,), d, jnp.int32) for d in range(64)]

        def tile_loop(j, carry, c=c):
            r = _RC * c + j
            rows = [j * _SB + 16 * k + lane for k in range(8)]
            hs = {}
            for ti in range(8):
                ov = outa_v if ti % 2 == 0 else outb_v
                if ti >= 2:
                    hs[ti - 2].wait()
                for dr in range(8):
                    col = cols[8 * ti + dr]
                    vs = [
                        plsc.load_gather(rows_v, [rows[k], col])
                        for k in range(8)
                    ]
                    for k in range(8):
                        ov[dr, pl.ds(16 * k, 16)] = vs[k]
                hs[ti] = pltpu.async_copy(
                    ov,
                    out_hbm.at[r * 256 + ti * 32 + w],
                    wsem_a if ti % 2 == 0 else wsem_b,
                )
            hs[6].wait()
            hs[7].wait()
            return carry

        lax.fori_loop(0, _RC, tile_loop, 0)


def kernel(x, weight):
    xt = x.T
    mesh = plsc.VectorSubcoreMesh(core_axis_name="c", subcore_axis_name="s")
    k = pl.kernel(
        _embed_body,
        mesh=mesh,
        out_type=jax.ShapeDtypeStruct((_R * 8 * _NW, 8, 128), jnp.float32),
        scratch_types=[
            pltpu.VMEM((_CHUNK,), jnp.int32),
            pltpu.VMEM((_CHUNK, _D), jnp.float32),
            pltpu.VMEM((8, 128), jnp.float32),
            pltpu.VMEM((8, 128), jnp.float32),
            pltpu.SemaphoreType.DMA,
            pltpu.SemaphoreType.DMA,
            pltpu.SemaphoreType.DMA,
            pltpu.SemaphoreType.DMA,
        ],
        compiler_params=pltpu.CompilerParams(
            use_tc_tiling_on_sc=False, needs_layout_passes=False
        ),
    )
    out3 = k(xt, weight)
    t = out3.reshape(_R, 8, _NW, 8, 128)
    return t.transpose(2, 4, 0, 1, 3).reshape(_S, _R, _D)


# speedup vs baseline: 1.9988x; 1.9988x over previous
import jax
import jax.numpy as jnp
from jax import lax
from jax.experimental import pallas as pl
from jax.experimental.pallas import tpu as pltpu
from jax.experimental.pallas import tpu_sc as plsc

_S = 4096
_R = 26
_D = 64
_NC, _NS = 2, 16
_NW = _NC * _NS
_SB = _S // _NW
_RC = 13
_CHUNK = _RC * _SB


def _embed_body(xt_hbm, table_hbm, out_hbm, idx_v, rows_v, big_v,
                gsem, wsem_a, wsem_b, isem):
    w = lax.axis_index("s") * _NC + lax.axis_index("c")
    s0 = w * _SB
    lane = jax.lax.iota(jnp.int32, 16)
    dstc = [16 * m + lane for m in range(8)]

    for c in range(2):
        ihs = [
            pltpu.async_copy(
                xt_hbm.at[_RC * c + j, pl.ds(s0, _SB)],
                idx_v.at[pl.ds(_SB * j, _SB)],
                isem,
            )
            for j in range(_RC)
        ]
        for h in ihs:
            h.wait()
        pltpu.async_copy(table_hbm.at[idx_v], rows_v, gsem).wait()

        def tile_loop(j, carry, c=c):
            r = _RC * c + j
            rb = [j * _SB + 16 * m + lane for m in range(8)]
            for half in range(2):
                wsem = wsem_a if half == 0 else wsem_b

                @pl.when(j > 0)
                def _(half=half, wsem=wsem):
                    for ti in range(4 * half, 4 * half + 4):
                        pltpu.make_async_copy(
                            out_hbm.at[0],
                            big_v.at[pl.ds(8 * ti, 8), :],
                            wsem,
                        ).wait()

                for q in (2 * half, 2 * half + 1):
                    def diag_loop(i, cc, q=q, rb=rb):
                        dg = q * 16 + ((i + lane) & 15)
                        for m0 in range(0, 8, 4):
                            vs = [
                                plsc.load_gather(rows_v, [rb[m0 + t], dg])
                                for t in range(4)
                            ]
                            for t in range(4):
                                plsc.store_scatter(
                                    big_v, [dg, dstc[m0 + t]], vs[t]
                                )
                        return cc
                    lax.fori_loop(0, 16, diag_loop, 0)
                for ti in range(4 * half, 4 * half + 4):
                    pltpu.async_copy(
                        big_v.at[pl.ds(8 * ti, 8), :],
                        out_hbm.at[r * 256 + ti * 32 + w],
                        wsem,
                    )
            return carry

        lax.fori_loop(0, _RC, tile_loop, 0)
        for ti in range(8):
            pltpu.make_async_copy(
                out_hbm.at[0],
                big_v.at[pl.ds(8 * ti, 8), :],
                wsem_a if ti < 4 else wsem_b,
            ).wait()


def kernel(x, weight):
    xt = x.T
    mesh = plsc.VectorSubcoreMesh(core_axis_name="c", subcore_axis_name="s")
    k = pl.kernel(
        _embed_body,
        mesh=mesh,
        out_type=jax.ShapeDtypeStruct((_R * 8 * _NW, 8, 128), jnp.float32),
        scratch_types=[
            pltpu.VMEM((_CHUNK,), jnp.int32),
            pltpu.VMEM((_CHUNK, _D), jnp.float32),
            pltpu.VMEM((_D, 128), jnp.float32),
            pltpu.SemaphoreType.DMA,
            pltpu.SemaphoreType.DMA,
            pltpu.SemaphoreType.DMA,
            pltpu.SemaphoreType.DMA,
        ],
        compiler_params=pltpu.CompilerParams(
            use_tc_tiling_on_sc=False, needs_layout_passes=False
        ),
    )
    out3 = k(xt, weight)
    t = out3.reshape(_R, 8, _NW, 8, 128)
    return t.transpose(2, 4, 0, 1, 3).reshape(_S, _R, _D)
